# baseline (speedup 1.0000x reference)
import functools

import jax
import jax.numpy as jnp
from jax import lax
from jax.experimental import pallas as pl
from jax.experimental.pallas import tpu as pltpu
from jax.experimental.pallas import tpu_sc as plsc

_NC = 2
_NS = 16
_NW = _NC * _NS
_L = 16


def _sc_embed_sum(xi, table, B, HIST, D):
    CH = HIST // 2
    b_per_w = B // _NW
    nd = D // _L
    mesh = plsc.VectorSubcoreMesh(core_axis_name="c", subcore_axis_name="s")

    @functools.partial(
        pl.kernel,
        out_type=jax.ShapeDtypeStruct((B, D), jnp.float32),
        mesh=mesh,
        scratch_types=[
            pltpu.VMEM((2 * b_per_w, CH), jnp.int32),
            pltpu.VMEM((HIST, D), jnp.float32),
            pltpu.VMEM((b_per_w, D), jnp.float32),
            pltpu.SemaphoreType.DMA,
        ],
        compiler_params=pltpu.CompilerParams(use_tc_tiling_on_sc=False),
    )
    def k(x_hbm, tab_hbm, out_hbm, idx_v, rows_v, sums_v, sem):
        wid = lax.axis_index("s") * _NC + lax.axis_index("c")
        pltpu.sync_copy(x_hbm.at[pl.ds(wid * 2 * b_per_w, 2 * b_per_w)], idx_v)

        @pl.loop(0, b_per_w)
        def _(r):
            c0 = pltpu.async_copy(
                tab_hbm.at[idx_v.at[2 * r]], rows_v.at[pl.ds(0, CH)], sem)
            c1 = pltpu.async_copy(
                tab_hbm.at[idx_v.at[2 * r + 1]], rows_v.at[pl.ds(CH, CH)], sem)
            c0.wait()
            c1.wait()

            def body(h, accs):
                return tuple(
                    accs[d] + rows_v[h, pl.ds(d * _L, _L)] for d in range(nd))

            accs = lax.fori_loop(
                0, HIST, body,
                tuple(jnp.zeros((_L,), jnp.float32) for _ in range(nd)),
                unroll=4)
            for d in range(nd):
                sums_v[r, pl.ds(d * _L, _L)] = accs[d]

        pltpu.sync_copy(sums_v, out_hbm.at[pl.ds(wid * b_per_w, b_per_w)])

    return k(xi, table)


def _mlp(sums, W1, b1, W2, b2, HIST):
    B, D = sums.shape
    H = W1.shape[1]
    O = W2.shape[1]

    def mlp_body(s_ref, w1_ref, b1_ref, w2_ref, b2_ref, o_ref):
        xm = s_ref[...] * (1.0 / HIST)
        x1 = jnp.dot(xm, w1_ref[...], preferred_element_type=jnp.float32)
        a1 = jnp.maximum(x1 + b1_ref[...], 0.0)
        o_ref[...] = (
            jnp.dot(a1, w2_ref[...], preferred_element_type=jnp.float32)
            + b2_ref[...])

    return pl.pallas_call(
        mlp_body,
        out_shape=jax.ShapeDtypeStruct((B, O), jnp.float32),
    )(sums, W1, b1.reshape(1, H), W2, b2.reshape(1, O))


def kernel(x, table, W1, b1, W2, b2):
    B, HIST = x.shape
    _, D = table.shape
    xi = x.astype(jnp.int32).reshape(2 * B, HIST // 2)
    sums = _sc_embed_sum(xi, table, B, HIST, D)
    return _mlp(sums, W1, b1, W2, b2, HIST)
